# Initial kernel scaffold; baseline (speedup 1.0000x reference)
import jax
import jax.numpy as jnp
from jax.experimental import pallas as pl


def kernel(x, edge_index, W1, b1, W2, b2, W3, b3):
    raise NotImplementedError("write your pallas kernel here")


# speedup vs baseline: 8.0111x; 8.0111x over previous
import functools

import jax
import jax.numpy as jnp
from jax import lax
from jax.experimental import pallas as pl
from jax.experimental.pallas import tpu as pltpu
from jax.experimental.pallas import tpu_sc as plsc

N = 10000
E = 320000
D_IN = 128
D_HID = 256
D_OUT = 128

NPAD = 10240
NTILES = 16
RPT = NPAD // NTILES
CH = 160
CHD = 80
E_PAD = 16 * CH * 128
K = 128
CB = 40
NSTAGE = CH // CB


def _sc_mesh():
    return plsc.VectorSubcoreMesh(core_axis_name="c", subcore_axis_name="s")


def _sc_count(ones_hbm_arr, dst_c):
    @functools.partial(
        pl.kernel,
        out_type=jax.ShapeDtypeStruct((2, NPAD, 128), jnp.float32),
        mesh=_sc_mesh(),
        scratch_types=[
            pltpu.VMEM((CB, K), jnp.int32),
            pltpu.VMEM((K, 128), jnp.float32),
            pltpu.VMEM_SHARED((NPAD, 128), jnp.float32),
            pltpu.SemaphoreType.DMA,
        ],
    )
    def count_kernel(ones_hbm, dst_hbm, out_hbm, dst_v, ones_v, acc_sh, sem):
        c = lax.axis_index("c")
        s = lax.axis_index("s")
        base = s * RPT
        pltpu.sync_copy(ones_hbm.at[pl.ds(base, RPT)], acc_sh.at[pl.ds(base, RPT)])
        pltpu.sync_copy(ones_hbm.at[pl.ds(0, K)], ones_v)
        plsc.subcore_barrier()

        def stage(g, carry):
            pltpu.sync_copy(dst_hbm.at[c, s, pl.ds(g * CB, CB)], dst_v)
            for j in range(CB):
                pltpu.sync_copy(ones_v, acc_sh.at[dst_v.at[j]], add=True)
            return carry

        lax.fori_loop(0, CHD // CB, stage, 0)
        plsc.subcore_barrier()
        pltpu.sync_copy(acc_sh.at[pl.ds(base, RPT)], out_hbm.at[c, pl.ds(base, RPT)])

    return count_kernel(ones_hbm_arr, dst_c)


def _sc_aggregate(hp_flat, src_c, dst_c, nch):
    @functools.partial(
        pl.kernel,
        out_type=jax.ShapeDtypeStruct((2, NPAD, 128), jnp.float32),
        mesh=_sc_mesh(),
        scratch_types=[
            pltpu.VMEM((CB, K), jnp.int32),
            pltpu.VMEM((CB, K), jnp.int32),
            pltpu.VMEM((K, 128), jnp.float32),
            pltpu.VMEM((K, 128), jnp.float32),
            pltpu.VMEM_SHARED((NPAD, 128), jnp.float32),
            pltpu.SemaphoreType.DMA,
            pltpu.SemaphoreType.DMA,
        ],
    )
    def agg_kernel(hp_hbm, src_hbm, dst_hbm, out_hbm, src_v, dst_v,
                   rows_a, rows_b, acc_sh, sem_a, sem_b):
        c = lax.axis_index("c")
        s = lax.axis_index("s")
        base = s * RPT
        pltpu.sync_copy(hp_hbm.at[pl.ds(c * NPAD + base, RPT)],
                        acc_sh.at[pl.ds(base, RPT)])
        plsc.subcore_barrier()
        rows = (rows_a, rows_b)
        sems = (sem_a, sem_b)

        def stage(g, carry):
            pltpu.sync_copy(src_hbm.at[c, s, pl.ds(g * CB, CB)], src_v)
            pltpu.sync_copy(dst_hbm.at[c, s, pl.ds(g * CB, CB)], dst_v)
            pending = pltpu.async_copy(hp_hbm.at[src_v.at[0]], rows[0], sems[0])
            for j in range(CB):
                if j + 1 < CB:
                    nxt = pltpu.async_copy(
                        hp_hbm.at[src_v.at[j + 1]], rows[(j + 1) % 2], sems[(j + 1) % 2])
                pending.wait()
                pltpu.sync_copy(rows[j % 2], acc_sh.at[dst_v.at[j]], add=True)
                if j + 1 < CB:
                    pending = nxt
            return carry

        lax.fori_loop(0, nch // CB, stage, 0)
        plsc.subcore_barrier()
        pltpu.sync_copy(acc_sh.at[pl.ds(base, RPT)], out_hbm.at[c, pl.ds(base, RPT)])

    return agg_kernel(hp_flat, src_c, dst_c)


_MB = 1024


def _dinv_kernel(degp):
    def body(d_ref, o_ref):
        o_ref[...] = lax.rsqrt(d_ref[0] + d_ref[1] - 1.0)

    return pl.pallas_call(
        body,
        grid=(NPAD // _MB,),
        in_specs=[pl.BlockSpec((2, _MB, 128), lambda m: (0, m, 0))],
        out_specs=pl.BlockSpec((_MB, 128), lambda m: (m, 0)),
        out_shape=jax.ShapeDtypeStruct((NPAD, 128), jnp.float32),
    )(degp)


def _split_w(w):
    k, d = w.shape
    return jnp.stack([w[:, : d // 2], w[:, d // 2:]])


def _mm_first(xp, w1s, dinv):
    half = w1s.shape[2]

    def body(x_ref, w_ref, dinv_ref, o_ref):
        h = jnp.dot(x_ref[...], w_ref[0], preferred_element_type=jnp.float32,
                    precision=lax.Precision.HIGHEST)
        o_ref[0] = h * dinv_ref[...]

    return pl.pallas_call(
        body,
        grid=(2, NPAD // _MB),
        in_specs=[
            pl.BlockSpec((_MB, D_IN), lambda c, m: (m, 0)),
            pl.BlockSpec((1, D_IN, half), lambda c, m: (c, 0, 0)),
            pl.BlockSpec((_MB, 128), lambda c, m: (m, 0)),
        ],
        out_specs=pl.BlockSpec((1, _MB, half), lambda c, m: (c, m, 0)),
        out_shape=jax.ShapeDtypeStruct((2, NPAD, half), jnp.float32),
    )(xp, w1s, dinv)


def _mm_mid(s_prev, b_prev, ws, dinv):
    half = ws.shape[2]

    def body(s_ref, b_ref, w_ref, dinv_ref, o_ref):
        dinv = dinv_ref[...]
        h0 = jnp.maximum(s_ref[0] * dinv + b_ref[0, 0:128][None, :], 0.0)
        h1 = jnp.maximum(s_ref[1] * dinv + b_ref[0, 128:256][None, :], 0.0)
        acc = jnp.dot(h0, w_ref[0, 0:128, :], preferred_element_type=jnp.float32,
                      precision=lax.Precision.HIGHEST)
        acc = acc + jnp.dot(h1, w_ref[0, 128:256, :], preferred_element_type=jnp.float32,
                            precision=lax.Precision.HIGHEST)
        o_ref[0] = acc * dinv

    return pl.pallas_call(
        body,
        grid=(2, NPAD // _MB),
        in_specs=[
            pl.BlockSpec((2, _MB, 128), lambda c, m: (0, m, 0)),
            pl.BlockSpec((1, D_HID), lambda c, m: (0, 0)),
            pl.BlockSpec((1, D_HID, half), lambda c, m: (c, 0, 0)),
            pl.BlockSpec((_MB, 128), lambda c, m: (m, 0)),
        ],
        out_specs=pl.BlockSpec((1, _MB, half), lambda c, m: (c, m, 0)),
        out_shape=jax.ShapeDtypeStruct((2, NPAD, half), jnp.float32),
    )(s_prev, b_prev, ws, dinv)


def _mm_mid_full(s_prev, b_prev, w, dinv):
    def body(s_ref, b_ref, w_ref, dinv_ref, o_ref):
        dinv = dinv_ref[...]
        h0 = jnp.maximum(s_ref[0] * dinv + b_ref[0, 0:128][None, :], 0.0)
        h1 = jnp.maximum(s_ref[1] * dinv + b_ref[0, 128:256][None, :], 0.0)
        acc = jnp.dot(h0, w_ref[0:128, :], preferred_element_type=jnp.float32,
                      precision=lax.Precision.HIGHEST)
        acc = acc + jnp.dot(h1, w_ref[128:256, :], preferred_element_type=jnp.float32,
                            precision=lax.Precision.HIGHEST)
        o_ref[...] = acc * dinv

    return pl.pallas_call(
        body,
        grid=(NPAD // _MB,),
        in_specs=[
            pl.BlockSpec((2, _MB, 128), lambda m: (0, m, 0)),
            pl.BlockSpec((1, D_HID), lambda m: (0, 0)),
            pl.BlockSpec((D_HID, D_OUT), lambda m: (0, 0)),
            pl.BlockSpec((_MB, 128), lambda m: (m, 0)),
        ],
        out_specs=pl.BlockSpec((_MB, D_OUT), lambda m: (m, 0)),
        out_shape=jax.ShapeDtypeStruct((NPAD, D_OUT), jnp.float32),
    )(s_prev, b_prev, w, dinv)


def _mm_final(s3, hp3, b3, dinv):
    def body(s_ref, hp_ref, b_ref, dinv_ref, o_ref):
        o_ref[...] = (s_ref[0] + s_ref[1] - hp_ref[0]) * dinv_ref[...] \
            + b_ref[0][None, :]

    return pl.pallas_call(
        body,
        grid=(NPAD // _MB,),
        in_specs=[
            pl.BlockSpec((2, _MB, D_OUT), lambda m: (0, m, 0)),
            pl.BlockSpec((1, _MB, D_OUT), lambda m: (0, m, 0)),
            pl.BlockSpec((1, D_OUT), lambda m: (0, 0)),
            pl.BlockSpec((_MB, 128), lambda m: (m, 0)),
        ],
        out_specs=pl.BlockSpec((_MB, D_OUT), lambda m: (m, 0)),
        out_shape=jax.ShapeDtypeStruct((NPAD, D_OUT), jnp.float32),
    )(s3, hp3, b3, dinv)


def kernel(x, edge_index, W1, b1, W2, b2, W3, b3):
    src = edge_index[0]
    dst = edge_index[1]
    pad = E_PAD - E
    src_p = jnp.concatenate([src, jnp.zeros((pad,), jnp.int32)])
    dst_p = jnp.concatenate([dst, jnp.full((pad,), N, jnp.int32)])
    dst_t = dst_p.reshape(NTILES, CH, K)
    src_t = src_p.reshape(NTILES, CH, K)
    src_c12 = jnp.stack([src_t, src_t + NPAD])
    dst_c12 = jnp.stack([dst_t, dst_t])
    src_p3 = src_p.reshape(2, NTILES, CHD, K)
    src_c3 = jnp.stack([src_p3[0], src_p3[1] + NPAD])
    dst_c3 = dst_p.reshape(2, NTILES, CHD, K)
    xp = jnp.concatenate([x, jnp.zeros((NPAD - N, D_IN), jnp.float32)])

    ones_arr = jnp.ones((NPAD, 128), jnp.float32)
    degp = _sc_count(ones_arr, dst_c3)
    dinv = _dinv_kernel(degp)
    hp1 = _mm_first(xp, _split_w(W1), dinv)
    s1 = _sc_aggregate(hp1.reshape(2 * NPAD, 128), src_c12, dst_c12, CH)
    hp2 = _mm_mid(s1, b1.reshape(1, D_HID), _split_w(W2), dinv)
    s2 = _sc_aggregate(hp2.reshape(2 * NPAD, 128), src_c12, dst_c12, CH)
    hp3d = _mm_mid(s2, b2.reshape(1, D_HID), jnp.stack([W3, W3]), dinv)
    s3 = _sc_aggregate(hp3d.reshape(2 * NPAD, 128), src_c3, dst_c3, CHD)
    out = _mm_final(s3, hp3d, b3.reshape(1, D_OUT), dinv)
    return out[:N]


# speedup vs baseline: 8.8749x; 1.1078x over previous
import functools

import jax
import jax.numpy as jnp
from jax import lax
from jax.experimental import pallas as pl
from jax.experimental.pallas import tpu as pltpu
from jax.experimental.pallas import tpu_sc as plsc

N = 10000
E = 320000
D_IN = 128
D_HID = 256
D_OUT = 128

NPAD = 10240
NTILES = 16
RPT = NPAD // NTILES
CH = 160
CHD = 80
E_PAD = 16 * CH * 128
K = 128
CB = 40
NSTAGE = CH // CB


def _sc_mesh():
    return plsc.VectorSubcoreMesh(core_axis_name="c", subcore_axis_name="s")


DEGW = 16


def _sc_count(ones_hbm_arr, dst_c):
    @functools.partial(
        pl.kernel,
        out_type=jax.ShapeDtypeStruct((2, NPAD, DEGW), jnp.float32),
        mesh=_sc_mesh(),
        scratch_types=[
            pltpu.VMEM((CB, K), jnp.int32),
            pltpu.VMEM((K, DEGW), jnp.float32),
            pltpu.VMEM_SHARED((NPAD, DEGW), jnp.float32),
            pltpu.SemaphoreType.DMA,
        ],
    )
    def count_kernel(ones_hbm, dst_hbm, out_hbm, dst_v, ones_v, acc_sh, sem):
        c = lax.axis_index("c")
        s = lax.axis_index("s")
        base = s * RPT
        pltpu.sync_copy(ones_hbm.at[pl.ds(base, RPT)], acc_sh.at[pl.ds(base, RPT)])
        pltpu.sync_copy(ones_hbm.at[pl.ds(0, K)], ones_v)
        plsc.subcore_barrier()

        def stage(g, carry):
            pltpu.sync_copy(dst_hbm.at[c, s, pl.ds(g * CB, CB)], dst_v)
            for j in range(CB):
                pltpu.sync_copy(ones_v, acc_sh.at[dst_v.at[j]], add=True)
            return carry

        lax.fori_loop(0, CHD // CB, stage, 0)
        plsc.subcore_barrier()
        pltpu.sync_copy(acc_sh.at[pl.ds(base, RPT)], out_hbm.at[c, pl.ds(base, RPT)])

    return count_kernel(ones_hbm_arr, dst_c)


def _sc_aggregate(hp_flat, src_c, dst_c, nch):
    @functools.partial(
        pl.kernel,
        out_type=jax.ShapeDtypeStruct((2, NPAD, 128), jnp.float32),
        mesh=_sc_mesh(),
        scratch_types=[
            pltpu.VMEM((CB, K), jnp.int32),
            pltpu.VMEM((CB, K), jnp.int32),
            pltpu.VMEM((K, 128), jnp.float32),
            pltpu.VMEM((K, 128), jnp.float32),
            pltpu.VMEM_SHARED((NPAD, 128), jnp.float32),
            pltpu.SemaphoreType.DMA,
            pltpu.SemaphoreType.DMA,
        ],
    )
    def agg_kernel(hp_hbm, src_hbm, dst_hbm, out_hbm, src_v, dst_v,
                   rows_a, rows_b, acc_sh, sem_a, sem_b):
        c = lax.axis_index("c")
        s = lax.axis_index("s")
        base = s * RPT
        pltpu.sync_copy(hp_hbm.at[pl.ds(c * NPAD + base, RPT)],
                        acc_sh.at[pl.ds(base, RPT)])
        plsc.subcore_barrier()
        rows = (rows_a, rows_b)
        sems = (sem_a, sem_b)

        def stage(g, carry):
            pltpu.sync_copy(src_hbm.at[c, s, pl.ds(g * CB, CB)], src_v)
            pltpu.sync_copy(dst_hbm.at[c, s, pl.ds(g * CB, CB)], dst_v)
            pending = pltpu.async_copy(hp_hbm.at[src_v.at[0]], rows[0], sems[0])
            for j in range(CB):
                if j + 1 < CB:
                    nxt = pltpu.async_copy(
                        hp_hbm.at[src_v.at[j + 1]], rows[(j + 1) % 2], sems[(j + 1) % 2])
                pending.wait()
                pltpu.sync_copy(rows[j % 2], acc_sh.at[dst_v.at[j]], add=True)
                if j + 1 < CB:
                    pending = nxt
            return carry

        lax.fori_loop(0, nch // CB, stage, 0)
        plsc.subcore_barrier()
        pltpu.sync_copy(acc_sh.at[pl.ds(base, RPT)], out_hbm.at[c, pl.ds(base, RPT)])

    return agg_kernel(hp_flat, src_c, dst_c)


_MB = 1024


def _split_w(w):
    k, d = w.shape
    return jnp.stack([w[:, : d // 2], w[:, d // 2:]])


def _mm_first(xp, w1s, degp):
    half = w1s.shape[2]

    def body(x_ref, w_ref, deg_ref, o_ref, dinv_ref):
        d = deg_ref[0, :, 0:1] + deg_ref[1, :, 0:1] - 1.0
        dv = lax.rsqrt(d)
        dinv_ref[...] = jnp.broadcast_to(dv, dinv_ref.shape)
        h = jnp.dot(x_ref[...], w_ref[0], preferred_element_type=jnp.float32,
                    precision=lax.Precision.HIGHEST)
        o_ref[0] = h * dv

    return pl.pallas_call(
        body,
        grid=(2, NPAD // _MB),
        in_specs=[
            pl.BlockSpec((_MB, D_IN), lambda c, m: (m, 0)),
            pl.BlockSpec((1, D_IN, half), lambda c, m: (c, 0, 0)),
            pl.BlockSpec((2, _MB, DEGW), lambda c, m: (0, m, 0)),
        ],
        out_specs=[
            pl.BlockSpec((1, _MB, half), lambda c, m: (c, m, 0)),
            pl.BlockSpec((_MB, 128), lambda c, m: (m, 0)),
        ],
        out_shape=[
            jax.ShapeDtypeStruct((2, NPAD, half), jnp.float32),
            jax.ShapeDtypeStruct((NPAD, 128), jnp.float32),
        ],
    )(xp, w1s, degp)


def _mm_mid(s_prev, b_prev, ws, dinv):
    half = ws.shape[2]

    def body(s_ref, b_ref, w_ref, dinv_ref, o_ref):
        dinv = dinv_ref[...]
        h0 = jnp.maximum(s_ref[0] * dinv + b_ref[0, 0:128][None, :], 0.0)
        h1 = jnp.maximum(s_ref[1] * dinv + b_ref[0, 128:256][None, :], 0.0)
        acc = jnp.dot(h0, w_ref[0, 0:128, :], preferred_element_type=jnp.float32,
                      precision=lax.Precision.HIGHEST)
        acc = acc + jnp.dot(h1, w_ref[0, 128:256, :], preferred_element_type=jnp.float32,
                            precision=lax.Precision.HIGHEST)
        o_ref[0] = acc * dinv

    return pl.pallas_call(
        body,
        grid=(2, NPAD // _MB),
        in_specs=[
            pl.BlockSpec((2, _MB, 128), lambda c, m: (0, m, 0)),
            pl.BlockSpec((1, D_HID), lambda c, m: (0, 0)),
            pl.BlockSpec((1, D_HID, half), lambda c, m: (c, 0, 0)),
            pl.BlockSpec((_MB, 128), lambda c, m: (m, 0)),
        ],
        out_specs=pl.BlockSpec((1, _MB, half), lambda c, m: (c, m, 0)),
        out_shape=jax.ShapeDtypeStruct((2, NPAD, half), jnp.float32),
    )(s_prev, b_prev, ws, dinv)


def _mm_mid_full(s_prev, b_prev, w, dinv):
    def body(s_ref, b_ref, w_ref, dinv_ref, o_ref):
        dinv = dinv_ref[...]
        h0 = jnp.maximum(s_ref[0] * dinv + b_ref[0, 0:128][None, :], 0.0)
        h1 = jnp.maximum(s_ref[1] * dinv + b_ref[0, 128:256][None, :], 0.0)
        acc = jnp.dot(h0, w_ref[0:128, :], preferred_element_type=jnp.float32,
                      precision=lax.Precision.HIGHEST)
        acc = acc + jnp.dot(h1, w_ref[128:256, :], preferred_element_type=jnp.float32,
                            precision=lax.Precision.HIGHEST)
        o_ref[...] = acc * dinv

    return pl.pallas_call(
        body,
        grid=(NPAD // _MB,),
        in_specs=[
            pl.BlockSpec((2, _MB, 128), lambda m: (0, m, 0)),
            pl.BlockSpec((1, D_HID), lambda m: (0, 0)),
            pl.BlockSpec((D_HID, D_OUT), lambda m: (0, 0)),
            pl.BlockSpec((_MB, 128), lambda m: (m, 0)),
        ],
        out_specs=pl.BlockSpec((_MB, D_OUT), lambda m: (m, 0)),
        out_shape=jax.ShapeDtypeStruct((NPAD, D_OUT), jnp.float32),
    )(s_prev, b_prev, w, dinv)


def _mm_final(s3, hp3, b3, dinv):
    def body(s_ref, hp_ref, b_ref, dinv_ref, o_ref):
        o_ref[...] = (s_ref[0] + s_ref[1] - hp_ref[0]) * dinv_ref[...] \
            + b_ref[0][None, :]

    return pl.pallas_call(
        body,
        grid=(NPAD // _MB,),
        in_specs=[
            pl.BlockSpec((2, _MB, D_OUT), lambda m: (0, m, 0)),
            pl.BlockSpec((1, _MB, D_OUT), lambda m: (0, m, 0)),
            pl.BlockSpec((1, D_OUT), lambda m: (0, 0)),
            pl.BlockSpec((_MB, 128), lambda m: (m, 0)),
        ],
        out_specs=pl.BlockSpec((_MB, D_OUT), lambda m: (m, 0)),
        out_shape=jax.ShapeDtypeStruct((NPAD, D_OUT), jnp.float32),
    )(s3, hp3, b3, dinv)


def kernel(x, edge_index, W1, b1, W2, b2, W3, b3):
    src = edge_index[0]
    dst = edge_index[1]
    pad = E_PAD - E
    src_p = jnp.concatenate([src, jnp.zeros((pad,), jnp.int32)])
    dst_p = jnp.concatenate([dst, jnp.full((pad,), N, jnp.int32)])
    dst_t = dst_p.reshape(NTILES, CH, K)
    src_t = src_p.reshape(NTILES, CH, K)
    src_c12 = jnp.stack([src_t, src_t + NPAD])
    dst_c12 = jnp.stack([dst_t, dst_t])
    src_p3 = src_p.reshape(2, NTILES, CHD, K)
    src_c3 = jnp.stack([src_p3[0], src_p3[1] + NPAD])
    dst_c3 = dst_p.reshape(2, NTILES, CHD, K)
    xp = jnp.concatenate([x, jnp.zeros((NPAD - N, D_IN), jnp.float32)])

    ones_arr = jnp.ones((NPAD, DEGW), jnp.float32)
    degp = _sc_count(ones_arr, dst_c3)
    hp1, dinv = _mm_first(xp, _split_w(W1), degp)
    s1 = _sc_aggregate(hp1.reshape(2 * NPAD, 128), src_c12, dst_c12, CH)
    hp2 = _mm_mid(s1, b1.reshape(1, D_HID), _split_w(W2), dinv)
    s2 = _sc_aggregate(hp2.reshape(2 * NPAD, 128), src_c12, dst_c12, CH)
    hp3d = _mm_mid(s2, b2.reshape(1, D_HID), jnp.stack([W3, W3]), dinv)
    s3 = _sc_aggregate(hp3d.reshape(2 * NPAD, 128), src_c3, dst_c3, CHD)
    out = _mm_final(s3, hp3d, b3.reshape(1, D_OUT), dinv)
    return out[:N]
